# baseline (speedup 1.0000x reference)
import jax
import jax.numpy as jnp
from jax.experimental import pallas as pl

_C = 192
_H = 32
_W = 32
_N = _H * _W
_OH = 30
_NEG = -3.0e38
_OFFS = tuple(di * _W + dj for di in range(3) for dj in range(3))


def _shl(v, o, rows):
    if o == 0:
        return v
    return jnp.concatenate(
        [jax.lax.slice(v, (0, o), (rows, _N)), jnp.zeros((rows, o), v.dtype)],
        axis=1)


def _shr(v, o, rows):
    if o == 0:
        return v
    return jnp.concatenate(
        [jnp.zeros((rows, o), v.dtype), jax.lax.slice(v, (0, 0), (rows, _N - o))],
        axis=1)


def _match_one(f1, f2):

    n1 = jnp.sqrt(jnp.sum(f1 * f1, axis=0, keepdims=True))
    fi = f1 / jnp.maximum(n1, 1e-12)
    n2 = jnp.sqrt(jnp.sum(f2 * f2, axis=0, keepdims=True))
    fr = f2 / jnp.maximum(n2, 1e-12)

    sqi = jnp.sum(fi * fi, axis=0, keepdims=True)
    sqr = jnp.sum(fr * fr, axis=0, keepdims=True)

    rn2 = _shl(sqr, _OFFS[0], 1)
    in2 = _shl(sqi, _OFFS[0], 1)
    for o in _OFFS[1:]:
        rn2 = rn2 + _shl(sqr, o, 1)
        in2 = in2 + _shl(sqi, o, 1)
    rn = jnp.sqrt(rn2) + 1e-5

    colx = jax.lax.broadcasted_iota(jnp.int32, (1, _N), 1)
    colvalid = ((colx % _W) < _OH) & ((colx // _W) < _OH)
    a_cat = jnp.concatenate(
        [_shl(fi, o, _C) for o in _OFFS]
        + [jnp.ones((1, _N), jnp.float32)], axis=0)
    b_cat = jnp.concatenate(
        [_shl(fr, o, _C) / rn for o in _OFFS]
        + [jnp.where(colvalid, 0.0, _NEG)], axis=0)

    _NH = _N // 2
    maxvs = []
    idxs = []
    for h in range(2):
        bh = jax.lax.slice(b_cat, (0, h * _NH), (9 * _C + 1, (h + 1) * _NH))
        acch = jax.lax.dot_general(
            a_cat, bh, (((0,), (0,)), ((), ())),
            preferred_element_type=jnp.float32)
        maxvs.append(jnp.max(acch, axis=1, keepdims=True))
        laneh = jax.lax.broadcasted_iota(jnp.int32, (_N, _NH), 1) + h * _NH
        idxs.append(jnp.min(
            jnp.where(acch == maxvs[h], laneh, jnp.int32(1 << 30)),
            axis=1, keepdims=True))
    use0 = maxvs[0] >= maxvs[1]
    maxval = jnp.where(use0, maxvs[0], maxvs[1])
    idx = jnp.where(use0, idxs[0], idxs[1])

    pack = jnp.concatenate(
        [(idx % _W).astype(jnp.float32),
         (idx // _W).astype(jnp.float32),
         maxval], axis=1)
    rows = jnp.transpose(pack)

    pxr = jax.lax.slice(rows, (0, 0), (1, _N))
    pyr = jax.lax.slice(rows, (1, 0), (2, _N))
    maxr = jax.lax.slice(rows, (2, 0), (3, _N))

    colxf = (colx % _W).astype(jnp.float32)
    colyf = (colx // _W).astype(jnp.float32)
    fxr = jnp.where(colvalid, pxr - colxf, 0.0)
    fyr = jnp.where(colvalid, pyr - colyf, 0.0)

    simr = jnp.where(colvalid, maxr / (jnp.sqrt(in2) + 1e-5), 0.0)

    off_rows = []
    for s in _OFFS:
        off_rows.append(_shr(fxr, s, 1))
        off_rows.append(_shr(fyr, s, 1))
    return jnp.concatenate(off_rows, axis=0), _shr(simr, _W + 1, 1)


def _match_kernel(f1_ref, f2_ref, off_ref, sim_ref):
    for b in range(2):
        off18, simsh = _match_one(f1_ref[b], f2_ref[b])
        off_ref[b] = off18
        sim_ref[b] = simsh


@jax.jit
def kernel(features1, features2):
    b = features1.shape[0]
    f1 = features1.reshape(b, _C, _N)
    f2 = features2.reshape(b, _C, _N)

    off, sim = pl.pallas_call(
        _match_kernel,
        in_specs=[
            pl.BlockSpec((b, _C, _N), lambda: (0, 0, 0)),
            pl.BlockSpec((b, _C, _N), lambda: (0, 0, 0)),
        ],
        out_specs=[
            pl.BlockSpec((b, 18, _N), lambda: (0, 0, 0)),
            pl.BlockSpec((b, 1, _N), lambda: (0, 0, 0)),
        ],
        out_shape=[
            jax.ShapeDtypeStruct((b, 18, _N), jnp.float32),
            jax.ShapeDtypeStruct((b, 1, _N), jnp.float32),
        ],
    )(f1, f2)

    off5 = off.reshape(b, 9, 2, _N).transpose(0, 1, 3, 2)
    pre_offset = off5.reshape(b, 9, _H, _W, 2)
    pre_flow = off5[:, 0].reshape(b, _H, _W, 2)
    pre_similarity = sim.reshape(b, 1, _H, _W)
    return (pre_flow, pre_offset, pre_similarity)


# speedup vs baseline: 1.0390x; 1.0390x over previous
import jax
import jax.numpy as jnp
from jax.experimental import pallas as pl

_C = 192
_H = 32
_W = 32
_N = _H * _W
_OH = 30
_NEG = -3.0e38
_OFFS = tuple(di * _W + dj for di in range(3) for dj in range(3))


def _shl(v, o, rows):
    if o == 0:
        return v
    return jnp.concatenate(
        [jax.lax.slice(v, (0, o), (rows, _N)), jnp.zeros((rows, o), v.dtype)],
        axis=1)


def _shr(v, o, rows):
    if o == 0:
        return v
    return jnp.concatenate(
        [jnp.zeros((rows, o), v.dtype), jax.lax.slice(v, (0, 0), (rows, _N - o))],
        axis=1)


def _match_kernel(f1_ref, f2_ref, off_ref, sim_ref):
    f1 = f1_ref[0]
    f2 = f2_ref[0]

    n1 = jnp.sqrt(jnp.sum(f1 * f1, axis=0, keepdims=True))
    fi = f1 / jnp.maximum(n1, 1e-12)
    n2 = jnp.sqrt(jnp.sum(f2 * f2, axis=0, keepdims=True))
    fr = f2 / jnp.maximum(n2, 1e-12)

    sqi = jnp.square(n1 / jnp.maximum(n1, 1e-12))
    sqr = jnp.sum(fr * fr, axis=0, keepdims=True)

    rn2 = _shl(sqr, _OFFS[0], 1)
    in2 = _shl(sqi, _OFFS[0], 1)
    for o in _OFFS[1:]:
        rn2 = rn2 + _shl(sqr, o, 1)
        in2 = in2 + _shl(sqi, o, 1)
    rn = jnp.sqrt(rn2) + 1e-5

    colx = jax.lax.broadcasted_iota(jnp.int32, (1, _N), 1)
    colvalid = ((colx % _W) < _OH) & ((colx // _W) < _OH)
    a_cat = jnp.concatenate(
        [_shl(fi, o, _C) for o in _OFFS]
        + [jnp.ones((1, _N), jnp.float32)], axis=0)
    b_cat = jnp.concatenate(
        [_shl(fr, o, _C) / rn for o in _OFFS]
        + [jnp.where(colvalid, 0.0, _NEG)], axis=0)

    _NH = _N // 2
    maxvs = []
    idxs = []
    for h in range(2):
        bh = jax.lax.slice(b_cat, (0, h * _NH), (9 * _C + 1, (h + 1) * _NH))
        acch = jax.lax.dot_general(
            a_cat, bh, (((0,), (0,)), ((), ())),
            preferred_element_type=jnp.float32)
        maxvs.append(jnp.max(acch, axis=1, keepdims=True))
        laneh = jax.lax.broadcasted_iota(jnp.int32, (_N, _NH), 1) + h * _NH
        idxs.append(jnp.min(
            jnp.where(acch == maxvs[h], laneh, jnp.int32(1 << 30)),
            axis=1, keepdims=True))
    use0 = maxvs[0] >= maxvs[1]
    maxval = jnp.where(use0, maxvs[0], maxvs[1])
    idx = jnp.where(use0, idxs[0], idxs[1])

    pack = jnp.concatenate(
        [(idx % _W).astype(jnp.float32),
         (idx // _W).astype(jnp.float32),
         maxval], axis=1)
    rows = jnp.transpose(pack)

    pxr = jax.lax.slice(rows, (0, 0), (1, _N))
    pyr = jax.lax.slice(rows, (1, 0), (2, _N))
    maxr = jax.lax.slice(rows, (2, 0), (3, _N))

    colxf = (colx % _W).astype(jnp.float32)
    colyf = (colx // _W).astype(jnp.float32)
    fxr = jnp.where(colvalid, pxr - colxf, 0.0)
    fyr = jnp.where(colvalid, pyr - colyf, 0.0)

    simr = jnp.where(colvalid, maxr / (jnp.sqrt(in2) + 1e-5), 0.0)

    off_rows = []
    for s in _OFFS:
        off_rows.append(_shr(fxr, s, 1))
        off_rows.append(_shr(fyr, s, 1))
    off_ref[0] = jnp.concatenate(off_rows, axis=0)

    sim_ref[0] = _shr(simr, _W + 1, 1)


@jax.jit
def kernel(features1, features2):
    b = features1.shape[0]
    f1 = features1.reshape(b, _C, _N)
    f2 = features2.reshape(b, _C, _N)

    off, sim = pl.pallas_call(
        _match_kernel,
        grid=(b,),
        in_specs=[
            pl.BlockSpec((1, _C, _N), lambda i: (i, 0, 0)),
            pl.BlockSpec((1, _C, _N), lambda i: (i, 0, 0)),
        ],
        out_specs=[
            pl.BlockSpec((1, 18, _N), lambda i: (i, 0, 0)),
            pl.BlockSpec((1, 1, _N), lambda i: (i, 0, 0)),
        ],
        out_shape=[
            jax.ShapeDtypeStruct((b, 18, _N), jnp.float32),
            jax.ShapeDtypeStruct((b, 1, _N), jnp.float32),
        ],
    )(f1, f2)

    off5 = off.reshape(b, 9, 2, _N).transpose(0, 1, 3, 2)
    pre_offset = off5.reshape(b, 9, _H, _W, 2)
    pre_flow = off5[:, 0].reshape(b, _H, _W, 2)
    pre_similarity = sim.reshape(b, 1, _H, _W)
    return (pre_flow, pre_offset, pre_similarity)
